# Initial kernel scaffold; baseline (speedup 1.0000x reference)
import jax
import jax.numpy as jnp
from jax.experimental import pallas as pl


def kernel(input_ids, embed, W0, W1):
    raise NotImplementedError("write your pallas kernel here")


# speedup vs baseline: 1.9986x; 1.9986x over previous
import functools

import jax
import jax.numpy as jnp
from jax import lax
from jax.experimental import pallas as pl
from jax.experimental.pallas import tpu as pltpu
from jax.experimental.pallas import tpu_sc as plsc

HIDDEN = 128
VOCAB_BLOCK = 1000
NUM_CORES = 2
NUM_SUBCORES = 16
NUM_WORKERS = NUM_CORES * NUM_SUBCORES
CHUNK = 128


def _fold_body(w0_ref, w1_ref, embed_ref, out_ref, wt_ref):
    @pl.when(pl.program_id(0) == 0)
    def _():
        wt_ref[...] = lax.dot_general(
            w0_ref[...], w1_ref[...], (((0,), (1,)), ((), ())),
            preferred_element_type=jnp.float32,
            precision=lax.Precision.HIGHEST,
        )

    out_ref[...] = jnp.dot(
        embed_ref[...], wt_ref[...],
        preferred_element_type=jnp.float32,
        precision=lax.Precision.HIGHEST,
    )


def _fold_table(embed, w0, w1):
    vocab = embed.shape[0]
    grid = vocab // VOCAB_BLOCK
    return pl.pallas_call(
        _fold_body,
        grid=(grid,),
        in_specs=[
            pl.BlockSpec((HIDDEN, HIDDEN), lambda i: (0, 0)),
            pl.BlockSpec((HIDDEN, HIDDEN), lambda i: (0, 0)),
            pl.BlockSpec((VOCAB_BLOCK, HIDDEN), lambda i: (i, 0)),
        ],
        out_specs=pl.BlockSpec((VOCAB_BLOCK, HIDDEN), lambda i: (i, 0)),
        out_shape=jax.ShapeDtypeStruct((vocab, HIDDEN), jnp.float32),
        scratch_shapes=[pltpu.VMEM((HIDDEN, HIDDEN), jnp.float32)],
    )(w0, w1, embed)


def _make_gather(n_ids):
    assert n_ids % (NUM_WORKERS * CHUNK) == 0
    b_per_w = n_ids // NUM_WORKERS
    n_chunks = b_per_w // CHUNK
    mesh = plsc.VectorSubcoreMesh(core_axis_name="c", subcore_axis_name="s")

    @functools.partial(
        pl.kernel,
        out_type=jax.ShapeDtypeStruct((n_ids, HIDDEN), jnp.float32),
        mesh=mesh,
        scratch_types=[
            pltpu.VMEM((b_per_w,), jnp.int32),
            pltpu.VMEM((CHUNK, HIDDEN), jnp.float32),
            pltpu.SemaphoreType.DMA,
        ],
    )
    def gather(table_hbm, idx_hbm, out_hbm, idx_v, rows_v, sem):
        wid = lax.axis_index("s") * NUM_CORES + lax.axis_index("c")
        base = wid * b_per_w
        pltpu.sync_copy(idx_hbm.at[pl.ds(base, b_per_w)], idx_v)

        def chunk_body(j, carry):
            off = j * CHUNK
            pltpu.async_copy(
                table_hbm.at[idx_v.at[pl.ds(off, CHUNK)]], rows_v, sem
            ).wait()
            pltpu.sync_copy(rows_v, out_hbm.at[pl.ds(base + off, CHUNK)])
            return carry

        lax.fori_loop(0, n_chunks, chunk_body, 0)

    return gather


def kernel(input_ids, embed, W0, W1):
    table2 = _fold_table(embed, W0, W1)
    ids = input_ids.reshape(-1).astype(jnp.int32)
    out = _make_gather(ids.shape[0])(table2, ids)
    return out.reshape(input_ids.shape + (HIDDEN,))


# speedup vs baseline: 2.3448x; 1.1732x over previous
import functools

import jax
import jax.numpy as jnp
from jax import lax
from jax.experimental import pallas as pl
from jax.experimental.pallas import tpu as pltpu
from jax.experimental.pallas import tpu_sc as plsc

HIDDEN = 128
VOCAB_BLOCK = 1000
NUM_CORES = 2
NUM_SUBCORES = 16
NUM_WORKERS = NUM_CORES * NUM_SUBCORES
CHUNK = 128


def _fold_body(w0_ref, w1_ref, embed_ref, out_ref, wt_ref):
    @pl.when(pl.program_id(0) == 0)
    def _():
        wt_ref[...] = lax.dot_general(
            w0_ref[...], w1_ref[...], (((0,), (1,)), ((), ())),
            preferred_element_type=jnp.float32,
            precision=lax.Precision.HIGHEST,
        )

    out_ref[...] = jnp.dot(
        embed_ref[...], wt_ref[...],
        preferred_element_type=jnp.float32,
        precision=lax.Precision.HIGHEST,
    )


def _fold_table(embed, w0, w1):
    vocab = embed.shape[0]
    grid = vocab // VOCAB_BLOCK
    return pl.pallas_call(
        _fold_body,
        grid=(grid,),
        in_specs=[
            pl.BlockSpec((HIDDEN, HIDDEN), lambda i: (0, 0)),
            pl.BlockSpec((HIDDEN, HIDDEN), lambda i: (0, 0)),
            pl.BlockSpec((VOCAB_BLOCK, HIDDEN), lambda i: (i, 0)),
        ],
        out_specs=pl.BlockSpec((VOCAB_BLOCK, HIDDEN), lambda i: (i, 0)),
        out_shape=jax.ShapeDtypeStruct((vocab, HIDDEN), jnp.float32),
        scratch_shapes=[pltpu.VMEM((HIDDEN, HIDDEN), jnp.float32)],
    )(w0, w1, embed)


SEQ_PAD = 56


def _make_gather(batch, seq):
    assert batch % NUM_WORKERS == 0
    b_per_w = batch // NUM_WORKERS
    n_idx_w = b_per_w * SEQ_PAD
    mesh = plsc.VectorSubcoreMesh(core_axis_name="c", subcore_axis_name="s")

    @functools.partial(
        pl.kernel,
        out_type=jax.ShapeDtypeStruct((batch, seq, HIDDEN), jnp.float32),
        mesh=mesh,
        scratch_types=[
            pltpu.VMEM((n_idx_w,), jnp.int32),
            pltpu.VMEM((seq, HIDDEN), jnp.float32),
            pltpu.SemaphoreType.DMA,
        ],
        compiler_params=pltpu.CompilerParams(use_tc_tiling_on_sc=True),
    )
    def gather(table_hbm, idx_hbm, out_hbm, idx_v, rows_v, sem):
        wid = lax.axis_index("s") * NUM_CORES + lax.axis_index("c")
        base_b = wid * b_per_w
        pltpu.sync_copy(idx_hbm.at[pl.ds(base_b * SEQ_PAD, n_idx_w)], idx_v)

        def batch_body(j, carry):
            pltpu.async_copy(
                table_hbm.at[idx_v.at[pl.ds(j * SEQ_PAD, seq)]], rows_v, sem
            ).wait()
            pltpu.sync_copy(rows_v, out_hbm.at[base_b + j])
            return carry

        lax.fori_loop(0, b_per_w, batch_body, 0)

    return gather


def kernel(input_ids, embed, W0, W1):
    table2 = _fold_table(embed, W0, W1)
    batch, seq = input_ids.shape
    ids = jnp.pad(input_ids.astype(jnp.int32), ((0, 0), (0, SEQ_PAD - seq)))
    out = _make_gather(batch, seq)(table2, ids.reshape(-1))
    return out


# speedup vs baseline: 2.8857x; 1.2307x over previous
import functools

import jax
import jax.numpy as jnp
from jax import lax
from jax.experimental import pallas as pl
from jax.experimental.pallas import tpu as pltpu
from jax.experimental.pallas import tpu_sc as plsc

HIDDEN = 128
VOCAB_BLOCK = 1000
NUM_CORES = 2
NUM_SUBCORES = 16
NUM_WORKERS = NUM_CORES * NUM_SUBCORES
CHUNK = 128


def _fold_body(w0_ref, w1_ref, embed_ref, out_ref, wt_ref):
    @pl.when(pl.program_id(0) == 0)
    def _():
        wt_ref[...] = lax.dot_general(
            w0_ref[...], w1_ref[...], (((0,), (1,)), ((), ())),
            preferred_element_type=jnp.float32,
            precision=lax.Precision.HIGHEST,
        )

    out_ref[...] = jnp.dot(
        embed_ref[...], wt_ref[...],
        preferred_element_type=jnp.float32,
        precision=lax.Precision.HIGHEST,
    )


def _fold_table(embed, w0, w1):
    vocab = embed.shape[0]
    grid = vocab // VOCAB_BLOCK
    return pl.pallas_call(
        _fold_body,
        grid=(grid,),
        in_specs=[
            pl.BlockSpec((HIDDEN, HIDDEN), lambda i: (0, 0)),
            pl.BlockSpec((HIDDEN, HIDDEN), lambda i: (0, 0)),
            pl.BlockSpec((VOCAB_BLOCK, HIDDEN), lambda i: (i, 0)),
        ],
        out_specs=pl.BlockSpec((VOCAB_BLOCK, HIDDEN), lambda i: (i, 0)),
        out_shape=jax.ShapeDtypeStruct((vocab, HIDDEN), jnp.float32),
        scratch_shapes=[pltpu.VMEM((HIDDEN, HIDDEN), jnp.float32)],
    )(w0, w1, embed)


SEQ_PAD = 56


def _make_gather(batch, seq):
    assert batch % NUM_WORKERS == 0
    b_per_w = batch // NUM_WORKERS
    n_idx_w = b_per_w * SEQ_PAD
    mesh = plsc.VectorSubcoreMesh(core_axis_name="c", subcore_axis_name="s")

    @functools.partial(
        pl.kernel,
        out_type=jax.ShapeDtypeStruct((batch, seq, HIDDEN), jnp.float32),
        mesh=mesh,
        scratch_types=[
            pltpu.VMEM((n_idx_w,), jnp.int32),
            pltpu.VMEM((seq, HIDDEN), jnp.float32),
            pltpu.VMEM((seq, HIDDEN), jnp.float32),
            pltpu.SemaphoreType.DMA,
            pltpu.SemaphoreType.DMA,
        ],
        compiler_params=pltpu.CompilerParams(use_tc_tiling_on_sc=True),
    )
    def gather(table_hbm, idx_hbm, out_hbm, idx_v, buf0, buf1, sem0, sem1):
        wid = lax.axis_index("s") * NUM_CORES + lax.axis_index("c")
        base_b = wid * b_per_w
        pltpu.sync_copy(idx_hbm.at[pl.ds(base_b * SEQ_PAD, n_idx_w)], idx_v)

        def g(j, buf, sem):
            return pltpu.make_async_copy(
                table_hbm.at[idx_v.at[pl.ds(j * SEQ_PAD, seq)]], buf, sem
            )

        g(0, buf0, sem0).start()

        def batch_body(j2, carry):
            j = 2 * j2
            g(j + 1, buf1, sem1).start()
            g(j, buf0, sem0).wait()
            pltpu.sync_copy(buf0, out_hbm.at[base_b + j])

            @pl.when(j + 2 < b_per_w)
            def _():
                g(j + 2, buf0, sem0).start()

            g(j + 1, buf1, sem1).wait()
            pltpu.sync_copy(buf1, out_hbm.at[base_b + j + 1])
            return carry

        lax.fori_loop(0, b_per_w // 2, batch_body, 0)

    return gather


def kernel(input_ids, embed, W0, W1):
    table2 = _fold_table(embed, W0, W1)
    batch, seq = input_ids.shape
    ids = jnp.pad(input_ids.astype(jnp.int32), ((0, 0), (0, SEQ_PAD - seq)))
    out = _make_gather(batch, seq)(table2, ids.reshape(-1))
    return out
